# baseline (speedup 1.0000x reference)
import jax
import jax.numpy as jnp
from jax.experimental import pallas as pl
from jax.experimental.pallas import tpu as pltpu

_N_SLICES = 4


def _copy_kernel(x_ref, o_ref, sems):
    n = x_ref.shape[0]
    rows = n // _N_SLICES
    copies = [
        pltpu.make_async_copy(
            x_ref.at[pl.ds(jnp.int32(i * rows), rows), :],
            o_ref.at[pl.ds(jnp.int32(i * rows), rows), :],
            sems.at[jnp.int32(i)],
        )
        for i in range(_N_SLICES)
    ]
    for c in copies:
        c.start()
    for c in copies:
        c.wait()


def kernel(x, edge_index):
    del edge_index
    n, d = x.shape
    return pl.pallas_call(
        _copy_kernel,
        in_specs=[pl.BlockSpec(memory_space=pltpu.MemorySpace.HBM)],
        out_specs=pl.BlockSpec(memory_space=pltpu.MemorySpace.HBM),
        out_shape=jax.ShapeDtypeStruct((n, d), x.dtype),
        scratch_shapes=[pltpu.SemaphoreType.DMA((_N_SLICES,))],
    )(x)


# speedup vs baseline: 23.9562x; 23.9562x over previous
import jax
import jax.numpy as jnp
from jax.experimental import pallas as pl
from jax.experimental.pallas import tpu as pltpu

_BLOCK_ROWS = 2000


def _copy_block(x_ref, o_ref):
    o_ref[...] = x_ref[...]


def kernel(x, edge_index):
    del edge_index
    n, d = x.shape
    return pl.pallas_call(
        _copy_block,
        grid=(n // _BLOCK_ROWS,),
        in_specs=[pl.BlockSpec((_BLOCK_ROWS, d), lambda i: (i, jnp.int32(0)))],
        out_specs=pl.BlockSpec((_BLOCK_ROWS, d), lambda i: (i, jnp.int32(0))),
        out_shape=jax.ShapeDtypeStruct((n, d), x.dtype),
    )(x)


# speedup vs baseline: 37.3435x; 1.5588x over previous
import jax
import jax.numpy as jnp
from jax.experimental import pallas as pl
from jax.experimental.pallas import tpu as pltpu

_BLOCK_ROWS = 5000


def _copy_block(x_ref, o_ref):
    o_ref[...] = x_ref[...]


def kernel(x, edge_index):
    del edge_index
    n, d = x.shape
    return pl.pallas_call(
        _copy_block,
        grid=(n // _BLOCK_ROWS,),
        in_specs=[pl.BlockSpec((_BLOCK_ROWS, d), lambda i: (i, jnp.int32(0)))],
        out_specs=pl.BlockSpec((_BLOCK_ROWS, d), lambda i: (i, jnp.int32(0))),
        out_shape=jax.ShapeDtypeStruct((n, d), x.dtype),
    )(x)
